# baseline (speedup 1.0000x reference)
import numpy as np
import jax
import jax.numpy as jnp
from jax.experimental import pallas as pl
from jax.experimental.pallas import tpu as pltpu

B = 128
EMB = 16
H = 96
STEPS = 4
OUT = 32
N = 81
E = 1620
EP = 1664


def _sudoku_edges():
    edges = set()
    for i in range(81):
        r, c = i // 9, i % 9
        for j in range(9):
            if j != c:
                edges.add((i, r * 9 + j))
            if j != r:
                edges.add((i, j * 9 + c))
        br, bc = 3 * (r // 3), 3 * (c // 3)
        for rr in range(br, br + 3):
            for cc in range(bc, bc + 3):
                j = rr * 9 + cc
                if j != i:
                    edges.add((i, j))
    e = np.array(sorted(edges), dtype=np.int32)
    return e[:, 0], e[:, 1]


_SRC, _DST = _sudoku_edges()

_GS = np.zeros((EP, N), np.float32)
_GS[np.arange(E), _SRC] = 1.0
_GD = np.zeros((EP, N), np.float32)
_GD[np.arange(E), _DST] = 1.0
_SCT = np.zeros((N, EP), np.float32)
_SCT[_DST, np.arange(E)] = 1.0

_CELL = np.arange(N)
_R_OH = np.zeros((N, 9), np.float32)
_R_OH[_CELL, _CELL // 9] = 1.0
_C_OH = np.zeros((N, 9), np.float32)
_C_OH[_CELL, _CELL % 9] = 1.0


def _pad_gates(w):
    parts = [jnp.pad(w[:, k * H:(k + 1) * H], ((0, 0), (0, 128 - H)))
             for k in range(4)]
    return jnp.concatenate(parts, axis=1)


def _fwd(pin_ref, m0_ref, rc_ref,
         w1_ref, b1_ref, w2_ref, b2_ref, w3_ref, b3_ref,
         gs_ref, gd_ref, sc_ref,
         wm1a_ref, wm1b_ref, bm1_ref, wm2_ref, bm2_ref, wm3_ref, bm3_ref,
         wm4_ref, bm4s_ref,
         wx_ref, wm_ref, wh_ref, owt_ref, ob_ref, out_ref):
    f32 = jnp.float32
    bf = jnp.bfloat16
    dot = lambda a, b: jnp.dot(a, b, preferred_element_type=f32)
    dotb = lambda a, b: jnp.dot(a.astype(bf), b, preferred_element_type=f32)

    pin = pin_ref[0]
    x = jnp.maximum(dot(pin, m0_ref[...]) + rc_ref[...], 0.0)
    x = jnp.maximum(dot(x, w1_ref[...]) + b1_ref[...], 0.0)
    x = jnp.maximum(dot(x, w2_ref[...]) + b2_ref[...], 0.0)
    x = dot(x, w3_ref[...]) + b3_ref[...]

    gs = gs_ref[...]
    gd = gd_ref[...]
    sc = sc_ref[...]

    h = x
    xb = x.astype(bf)
    hs = jnp.zeros((N, H), f32)
    c = jnp.zeros((N, H), f32)
    for step in range(STEPS):
        hb = h.astype(bf)
        a = jnp.dot(hb, wm1a_ref[...], preferred_element_type=f32).astype(bf)
        bv = jnp.dot(hb, wm1b_ref[...], preferred_element_type=f32).astype(bf)
        t = jnp.maximum(dot(gs, a) + dot(gd, bv) + bm1_ref[...], 0.0)
        t = jnp.maximum(dotb(t, wm2_ref[...]) + bm2_ref[...], 0.0)
        t = jnp.maximum(dotb(t, wm3_ref[...]) + bm3_ref[...], 0.0)
        seg = jnp.dot(sc, t.astype(bf), preferred_element_type=f32)
        m = dotb(seg, wm4_ref[...]) + bm4s_ref[...]

        gates = jnp.dot(xb, wx_ref[...], preferred_element_type=f32) \
            + dotb(m, wm_ref[...])
        if step > 0:
            gates = gates + dotb(hs, wh_ref[...])
        i_ = jax.nn.sigmoid(gates[:, 0:H])
        f_ = jax.nn.sigmoid(gates[:, 128:128 + H])
        g_ = jnp.tanh(gates[:, 256:256 + H])
        o_ = jax.nn.sigmoid(gates[:, 384:384 + H])
        c = f_ * c + i_ * g_
        hs = o_ * jnp.tanh(c)
        h = hs

    mp = jnp.max(h, axis=0, keepdims=True)
    logit = jnp.sum(h * mp, axis=1, keepdims=True) * (1.0 / H)
    logit = logit - jnp.max(logit, axis=0, keepdims=True)
    ew = jnp.exp(logit)
    aw = ew / jnp.sum(ew, axis=0, keepdims=True)
    pooled = jnp.sum(h * aw, axis=0, keepdims=True)
    out_ref[0] = dot(pooled, owt_ref[...]) + ob_ref[...]


def kernel(query, target, y_hat, digit_embed, row_embed, col_embed,
           in_layers, msg_layers, lstm_Wih, lstm_Whh, out_W, out_b,
           edge_index):
    f32 = jnp.float32
    prob = jnp.transpose(y_hat, (0, 2, 1))
    t_oh = jax.nn.one_hot(target, 10, dtype=f32)
    pin = jnp.concatenate([prob, t_oh], axis=-1)

    (w0, b0), (w1, b1), (w2, b2), (w3, b3) = in_layers
    m0 = jnp.concatenate([digit_embed @ w0[:, :EMB].T,
                          digit_embed @ w0[:, EMB:2 * EMB].T], axis=0)
    rows_e = jnp.asarray(_R_OH) @ row_embed
    cols_e = jnp.asarray(_C_OH) @ col_embed
    rc = (rows_e @ w0[:, 2 * EMB:3 * EMB].T
          + cols_e @ w0[:, 3 * EMB:].T + b0)

    bf = jnp.bfloat16
    (wm1, bm1), (wm2, bm2), (wm3, bm3), (wm4, bm4) = msg_layers
    wm1a = wm1[:, :H].T.astype(bf)
    wm1b = wm1[:, H:].T.astype(bf)
    bm4s = 20.0 * bm4

    wih_t = lstm_Wih.T
    wx = _pad_gates(wih_t[:H]).astype(bf)
    wm = _pad_gates(wih_t[H:]).astype(bf)
    wh = _pad_gates(lstm_Whh.T).astype(bf)

    row2 = lambda v: v[None, :]
    full = lambda arr: pl.BlockSpec(arr.shape, lambda b: (0,) * arr.ndim)

    ins = [pin, m0, rc,
           w1.T, row2(b1), w2.T, row2(b2), w3.T, row2(b3),
           jnp.asarray(_GS, dtype=bf), jnp.asarray(_GD, dtype=bf),
           jnp.asarray(_SCT, dtype=bf),
           wm1a, wm1b, row2(bm1), wm2.T.astype(bf), row2(bm2),
           wm3.T.astype(bf), row2(bm3), wm4.T.astype(bf), row2(bm4s),
           wx, wm, wh, out_W.T, row2(out_b)]

    in_specs = [pl.BlockSpec((1, N, 20), lambda b: (b, 0, 0))]
    in_specs += [full(a) for a in ins[1:]]

    out = pl.pallas_call(
        _fwd,
        grid=(B,),
        in_specs=in_specs,
        out_specs=pl.BlockSpec((1, 1, OUT), lambda b: (b, 0, 0)),
        out_shape=jax.ShapeDtypeStruct((B, 1, OUT), f32),
        compiler_params=pltpu.CompilerParams(
            dimension_semantics=("parallel",)),
    )(*ins)
    return out.reshape(B, OUT)


# speedup vs baseline: 2.1674x; 2.1674x over previous
import numpy as np
import jax
import jax.numpy as jnp
from jax.experimental import pallas as pl
from jax.experimental.pallas import tpu as pltpu

B = 128
EMB = 16
H = 96
STEPS = 4
OUT = 32
N = 81
E = 1620
EP = 1664


def _sudoku_edges():
    edges = set()
    for i in range(81):
        r, c = i // 9, i % 9
        for j in range(9):
            if j != c:
                edges.add((i, r * 9 + j))
            if j != r:
                edges.add((i, j * 9 + c))
        br, bc = 3 * (r // 3), 3 * (c // 3)
        for rr in range(br, br + 3):
            for cc in range(bc, bc + 3):
                j = rr * 9 + cc
                if j != i:
                    edges.add((i, j))
    e = np.array(sorted(edges), dtype=np.int32)
    return e[:, 0], e[:, 1]


_SRC, _DST = _sudoku_edges()

_GS = np.zeros((EP, N), np.float32)
_GS[np.arange(E), _SRC] = 1.0
_GD = np.zeros((EP, N), np.float32)
_GD[np.arange(E), _DST] = 1.0
_SCT = np.zeros((N, EP), np.float32)
_SCT[_DST, np.arange(E)] = 1.0

_CELL = np.arange(N)
_R_OH = np.zeros((N, 9), np.float32)
_R_OH[_CELL, _CELL // 9] = 1.0
_C_OH = np.zeros((N, 9), np.float32)
_C_OH[_CELL, _CELL % 9] = 1.0


def _pad_gates(w):
    parts = [jnp.pad(w[:, k * H:(k + 1) * H], ((0, 0), (0, 128 - H)))
             for k in range(4)]
    return jnp.concatenate(parts, axis=1)


PP = 4


def _fwd(pin_ref, m0_ref, rc_ref,
         w1_ref, b1_ref, w2_ref, b2_ref, w3_ref, b3_ref,
         gs_ref, gd_ref, sc_ref,
         wm1a_ref, wm1b_ref, bm1_ref, wm2_ref, bm2_ref, wm3_ref, bm3_ref,
         wm4_ref, bm4s_ref,
         wx_ref, wm_ref, wh_ref, owt_ref, ob_ref, out_ref):
    f32 = jnp.float32
    bf = jnp.bfloat16
    dot = lambda a, b: jnp.dot(a, b, preferred_element_type=f32)
    dotb = lambda a, b: jnp.dot(a.astype(bf), b, preferred_element_type=f32)
    R = range(PP)

    x = [pin_ref[p] for p in R]
    x = [jnp.maximum(dot(v, m0_ref[...]) + rc_ref[...], 0.0) for v in x]
    x = [jnp.maximum(dot(v, w1_ref[...]) + b1_ref[...], 0.0) for v in x]
    x = [jnp.maximum(dot(v, w2_ref[...]) + b2_ref[...], 0.0) for v in x]
    x = [dot(v, w3_ref[...]) + b3_ref[...] for v in x]

    gs = gs_ref[...]
    gd = gd_ref[...]
    sc = sc_ref[...]

    h = list(x)
    xb = [v.astype(bf) for v in x]
    hs = [jnp.zeros((N, H), f32) for p in R]
    c = [jnp.zeros((N, H), f32) for p in R]
    for step in range(STEPS):
        hb = [v.astype(bf) for v in h]
        a = [dot(v, wm1a_ref[...]).astype(bf) for v in hb]
        bv = [dot(v, wm1b_ref[...]).astype(bf) for v in hb]
        t = [jnp.maximum(dot(gs, a[p]) + dot(gd, bv[p]) + bm1_ref[...], 0.0)
             for p in R]
        t = [jnp.maximum(dotb(v, wm2_ref[...]) + bm2_ref[...], 0.0) for v in t]
        t = [jnp.maximum(dotb(v, wm3_ref[...]) + bm3_ref[...], 0.0) for v in t]
        seg = [jnp.dot(sc, v.astype(bf), preferred_element_type=f32) for v in t]
        m = [dotb(v, wm4_ref[...]) + bm4s_ref[...] for v in seg]

        gates = [jnp.dot(xb[p], wx_ref[...], preferred_element_type=f32)
                 + dotb(m[p], wm_ref[...]) for p in R]
        if step > 0:
            gates = [gates[p] + dotb(hs[p], wh_ref[...]) for p in R]
        i_ = [jax.nn.sigmoid(g[:, 0:H]) for g in gates]
        f_ = [jax.nn.sigmoid(g[:, 128:128 + H]) for g in gates]
        g_ = [jnp.tanh(g[:, 256:256 + H]) for g in gates]
        o_ = [jax.nn.sigmoid(g[:, 384:384 + H]) for g in gates]
        c = [f_[p] * c[p] + i_[p] * g_[p] for p in R]
        hs = [o_[p] * jnp.tanh(c[p]) for p in R]
        h = list(hs)

    mp = [jnp.max(v, axis=0, keepdims=True) for v in h]
    logit = [jnp.sum(h[p] * mp[p], axis=1, keepdims=True) * (1.0 / H)
             for p in R]
    logit = [v - jnp.max(v, axis=0, keepdims=True) for v in logit]
    ew = [jnp.exp(v) for v in logit]
    aw = [ew[p] / jnp.sum(ew[p], axis=0, keepdims=True) for p in R]
    pooled = [jnp.sum(h[p] * aw[p], axis=0, keepdims=True) for p in R]
    for p in R:
        out_ref[p] = dot(pooled[p], owt_ref[...]) + ob_ref[...]


def kernel(query, target, y_hat, digit_embed, row_embed, col_embed,
           in_layers, msg_layers, lstm_Wih, lstm_Whh, out_W, out_b,
           edge_index):
    f32 = jnp.float32
    prob = jnp.transpose(y_hat, (0, 2, 1))
    t_oh = jax.nn.one_hot(target, 10, dtype=f32)
    pin = jnp.concatenate([prob, t_oh], axis=-1)

    (w0, b0), (w1, b1), (w2, b2), (w3, b3) = in_layers
    m0 = jnp.concatenate([digit_embed @ w0[:, :EMB].T,
                          digit_embed @ w0[:, EMB:2 * EMB].T], axis=0)
    rows_e = jnp.asarray(_R_OH) @ row_embed
    cols_e = jnp.asarray(_C_OH) @ col_embed
    rc = (rows_e @ w0[:, 2 * EMB:3 * EMB].T
          + cols_e @ w0[:, 3 * EMB:].T + b0)

    bf = jnp.bfloat16
    (wm1, bm1), (wm2, bm2), (wm3, bm3), (wm4, bm4) = msg_layers
    wm1a = wm1[:, :H].T.astype(bf)
    wm1b = wm1[:, H:].T.astype(bf)
    bm4s = 20.0 * bm4

    wih_t = lstm_Wih.T
    wx = _pad_gates(wih_t[:H]).astype(bf)
    wm = _pad_gates(wih_t[H:]).astype(bf)
    wh = _pad_gates(lstm_Whh.T).astype(bf)

    row2 = lambda v: v[None, :]
    full = lambda arr: pl.BlockSpec(arr.shape, lambda b: (0,) * arr.ndim)

    ins = [pin, m0, rc,
           w1.T, row2(b1), w2.T, row2(b2), w3.T, row2(b3),
           jnp.asarray(_GS, dtype=bf), jnp.asarray(_GD, dtype=bf),
           jnp.asarray(_SCT, dtype=bf),
           wm1a, wm1b, row2(bm1), wm2.T.astype(bf), row2(bm2),
           wm3.T.astype(bf), row2(bm3), wm4.T.astype(bf), row2(bm4s),
           wx, wm, wh, out_W.T, row2(out_b)]

    in_specs = [pl.BlockSpec((PP, N, 20), lambda b: (b, 0, 0))]
    in_specs += [full(a) for a in ins[1:]]

    out = pl.pallas_call(
        _fwd,
        grid=(B // PP,),
        in_specs=in_specs,
        out_specs=pl.BlockSpec((PP, 1, OUT), lambda b: (b, 0, 0)),
        out_shape=jax.ShapeDtypeStruct((B, 1, OUT), f32),
        compiler_params=pltpu.CompilerParams(
            dimension_semantics=("parallel",)),
    )(*ins)
    return out.reshape(B, OUT)


# speedup vs baseline: 2.3961x; 1.1055x over previous
import numpy as np
import jax
import jax.numpy as jnp
from jax.experimental import pallas as pl
from jax.experimental.pallas import tpu as pltpu

B = 128
EMB = 16
H = 96
STEPS = 4
OUT = 32
N = 81
E = 1620
EP = 1664


def _sudoku_edges():
    edges = set()
    for i in range(81):
        r, c = i // 9, i % 9
        for j in range(9):
            if j != c:
                edges.add((i, r * 9 + j))
            if j != r:
                edges.add((i, j * 9 + c))
        br, bc = 3 * (r // 3), 3 * (c // 3)
        for rr in range(br, br + 3):
            for cc in range(bc, bc + 3):
                j = rr * 9 + cc
                if j != i:
                    edges.add((i, j))
    e = np.array(sorted(edges), dtype=np.int32)
    return e[:, 0], e[:, 1]


_SRC, _DST = _sudoku_edges()
_PERM = np.lexsort((_SRC, _DST))
_SRC_D = _SRC[_PERM]

_GS = np.zeros((E, N), np.float32)
_GS[np.arange(E), _SRC_D] = 1.0

_CELL = np.arange(N)
_R_OH = np.zeros((N, 9), np.float32)
_R_OH[_CELL, _CELL // 9] = 1.0
_C_OH = np.zeros((N, 9), np.float32)
_C_OH[_CELL, _CELL % 9] = 1.0


def _pad_gates(w):
    parts = [jnp.pad(w[:, k * H:(k + 1) * H], ((0, 0), (0, 128 - H)))
             for k in range(4)]
    return jnp.concatenate(parts, axis=1)


PP = 4


def _fwd(pin_ref, m0_ref, rc_ref,
         w1_ref, b1_ref, w2_ref, b2_ref, w3_ref, b3_ref,
         gs_ref,
         wm1a_ref, wm1b_ref, bm1_ref, wm2_ref, bm2_ref, wm3_ref, bm3_ref,
         wm4_ref, bm4s_ref,
         wx_ref, wm_ref, wh_ref, owt_ref, ob_ref, out_ref):
    f32 = jnp.float32
    bf = jnp.bfloat16
    dot = lambda a, b: jnp.dot(a, b, preferred_element_type=f32)
    dotb = lambda a, b: jnp.dot(a.astype(bf), b, preferred_element_type=f32)
    R = range(PP)

    x = [pin_ref[p] for p in R]
    x = [jnp.maximum(dot(v, m0_ref[...]) + rc_ref[...], 0.0) for v in x]
    x = [jnp.maximum(dot(v, w1_ref[...]) + b1_ref[...], 0.0) for v in x]
    x = [jnp.maximum(dot(v, w2_ref[...]) + b2_ref[...], 0.0) for v in x]
    x = [dot(v, w3_ref[...]) + b3_ref[...] for v in x]

    gs = gs_ref[...]

    h = list(x)
    xb = [v.astype(bf) for v in x]
    hs = [jnp.zeros((N, H), f32) for p in R]
    c = [jnp.zeros((N, H), f32) for p in R]
    for step in range(STEPS):
        hb = [v.astype(bf) for v in h]
        a = [dot(v, wm1a_ref[...]).astype(bf) for v in hb]
        bv = [dot(v, wm1b_ref[...]) + bm1_ref[...] for v in hb]
        t = [jnp.maximum(dot(gs, a[p]) + jnp.repeat(bv[p], 20, axis=0), 0.0)
             for p in R]
        t = [jnp.maximum(dotb(v, wm2_ref[...]) + bm2_ref[...], 0.0) for v in t]
        t = [jnp.maximum(dotb(v, wm3_ref[...]) + bm3_ref[...], 0.0) for v in t]
        seg = [jnp.sum(v.reshape(N, 20, H), axis=1) for v in t]
        m = [dotb(v, wm4_ref[...]) + bm4s_ref[...] for v in seg]

        gates = [jnp.dot(xb[p], wx_ref[...], preferred_element_type=f32)
                 + dotb(m[p], wm_ref[...]) for p in R]
        if step > 0:
            gates = [gates[p] + dotb(hs[p], wh_ref[...]) for p in R]
        i_ = [jax.nn.sigmoid(g[:, 0:H]) for g in gates]
        f_ = [jax.nn.sigmoid(g[:, 128:128 + H]) for g in gates]
        g_ = [jnp.tanh(g[:, 256:256 + H]) for g in gates]
        o_ = [jax.nn.sigmoid(g[:, 384:384 + H]) for g in gates]
        c = [f_[p] * c[p] + i_[p] * g_[p] for p in R]
        hs = [o_[p] * jnp.tanh(c[p]) for p in R]
        h = list(hs)

    mp = [jnp.max(v, axis=0, keepdims=True) for v in h]
    logit = [jnp.sum(h[p] * mp[p], axis=1, keepdims=True) * (1.0 / H)
             for p in R]
    logit = [v - jnp.max(v, axis=0, keepdims=True) for v in logit]
    ew = [jnp.exp(v) for v in logit]
    aw = [ew[p] / jnp.sum(ew[p], axis=0, keepdims=True) for p in R]
    pooled = [jnp.sum(h[p] * aw[p], axis=0, keepdims=True) for p in R]
    for p in R:
        out_ref[p] = dot(pooled[p], owt_ref[...]) + ob_ref[...]


def kernel(query, target, y_hat, digit_embed, row_embed, col_embed,
           in_layers, msg_layers, lstm_Wih, lstm_Whh, out_W, out_b,
           edge_index):
    f32 = jnp.float32
    prob = jnp.transpose(y_hat, (0, 2, 1))
    t_oh = jax.nn.one_hot(target, 10, dtype=f32)
    pin = jnp.concatenate([prob, t_oh], axis=-1)

    (w0, b0), (w1, b1), (w2, b2), (w3, b3) = in_layers
    m0 = jnp.concatenate([digit_embed @ w0[:, :EMB].T,
                          digit_embed @ w0[:, EMB:2 * EMB].T], axis=0)
    rows_e = jnp.asarray(_R_OH) @ row_embed
    cols_e = jnp.asarray(_C_OH) @ col_embed
    rc = (rows_e @ w0[:, 2 * EMB:3 * EMB].T
          + cols_e @ w0[:, 3 * EMB:].T + b0)

    bf = jnp.bfloat16
    (wm1, bm1), (wm2, bm2), (wm3, bm3), (wm4, bm4) = msg_layers
    wm1a = wm1[:, :H].T.astype(bf)
    wm1b = wm1[:, H:].T.astype(bf)
    bm4s = 20.0 * bm4

    wih_t = lstm_Wih.T
    wx = _pad_gates(wih_t[:H]).astype(bf)
    wm = _pad_gates(wih_t[H:]).astype(bf)
    wh = _pad_gates(lstm_Whh.T).astype(bf)

    row2 = lambda v: v[None, :]
    full = lambda arr: pl.BlockSpec(arr.shape, lambda b: (0,) * arr.ndim)

    ins = [pin, m0, rc,
           w1.T, row2(b1), w2.T, row2(b2), w3.T, row2(b3),
           jnp.asarray(_GS, dtype=bf),
           wm1a, wm1b, row2(bm1), wm2.T.astype(bf), row2(bm2),
           wm3.T.astype(bf), row2(bm3), wm4.T.astype(bf), row2(bm4s),
           wx, wm, wh, out_W.T, row2(out_b)]

    in_specs = [pl.BlockSpec((PP, N, 20), lambda b: (b, 0, 0))]
    in_specs += [full(a) for a in ins[1:]]

    out = pl.pallas_call(
        _fwd,
        grid=(B // PP,),
        in_specs=in_specs,
        out_specs=pl.BlockSpec((PP, 1, OUT), lambda b: (b, 0, 0)),
        out_shape=jax.ShapeDtypeStruct((B, 1, OUT), f32),
        compiler_params=pltpu.CompilerParams(
            dimension_semantics=("parallel",)),
    )(*ins)
    return out.reshape(B, OUT)


# speedup vs baseline: 2.5893x; 1.0807x over previous
import numpy as np
import jax
import jax.numpy as jnp
from jax.experimental import pallas as pl
from jax.experimental.pallas import tpu as pltpu

B = 128
EMB = 16
H = 96
STEPS = 4
OUT = 32
N = 81
E = 1620
EP = 1664


def _sudoku_edges():
    edges = set()
    for i in range(81):
        r, c = i // 9, i % 9
        for j in range(9):
            if j != c:
                edges.add((i, r * 9 + j))
            if j != r:
                edges.add((i, j * 9 + c))
        br, bc = 3 * (r // 3), 3 * (c // 3)
        for rr in range(br, br + 3):
            for cc in range(bc, bc + 3):
                j = rr * 9 + cc
                if j != i:
                    edges.add((i, j))
    e = np.array(sorted(edges), dtype=np.int32)
    return e[:, 0], e[:, 1]


_SRC, _DST = _sudoku_edges()
_PERM = np.lexsort((_SRC, _DST))
_SRC_D = _SRC[_PERM]

_GS = np.zeros((E, N), np.float32)
_GS[np.arange(E), _SRC_D] = 1.0
_SCT = np.zeros((N, E), np.float32)
_SCT[np.repeat(np.arange(N), 20), np.arange(E)] = 1.0

_CELL = np.arange(N)
_R_OH = np.zeros((N, 9), np.float32)
_R_OH[_CELL, _CELL // 9] = 1.0
_C_OH = np.zeros((N, 9), np.float32)
_C_OH[_CELL, _CELL % 9] = 1.0


def _pad_gates(w):
    parts = [jnp.pad(w[:, k * H:(k + 1) * H], ((0, 0), (0, 128 - H)))
             for k in range(4)]
    return jnp.concatenate(parts, axis=1)


PP = 4


def _fwd(pin_ref, m0_ref, rc_ref,
         w1_ref, b1_ref, w2_ref, b2_ref, w3_ref, b3_ref,
         gs_ref, sc_ref,
         wm1a_ref, wm1b_ref, bm1_ref, wm2_ref, bm2_ref, wm3_ref, bm3_ref,
         wm4_ref, bm4s_ref,
         wx_ref, wm_ref, wh_ref, owt_ref, ob_ref, out_ref):
    f32 = jnp.float32
    bf = jnp.bfloat16
    dot = lambda a, b: jnp.dot(a, b, preferred_element_type=f32)
    dotb = lambda a, b: jnp.dot(a.astype(bf), b, preferred_element_type=f32)
    R = range(PP)

    x = [pin_ref[p] for p in R]
    x = [jnp.maximum(dot(v, m0_ref[...]) + rc_ref[...], 0.0) for v in x]
    x = [jnp.maximum(dot(v, w1_ref[...]) + b1_ref[...], 0.0) for v in x]
    x = [jnp.maximum(dot(v, w2_ref[...]) + b2_ref[...], 0.0) for v in x]
    x = [dot(v, w3_ref[...]) + b3_ref[...] for v in x]

    gs = gs_ref[...]
    sc = sc_ref[...]

    h = list(x)
    xb = [v.astype(bf) for v in x]
    hs = [jnp.zeros((N, H), f32) for p in R]
    c = [jnp.zeros((N, H), f32) for p in R]
    for step in range(STEPS):
        hb = [v.astype(bf) for v in h]
        a = [dot(v, wm1a_ref[...]).astype(bf) for v in hb]
        bv = [dot(v, wm1b_ref[...]) + bm1_ref[...] for v in hb]
        t = [jnp.maximum(dot(gs, a[p]) + jnp.repeat(bv[p], 20, axis=0), 0.0)
             for p in R]
        t = [jnp.maximum(dotb(v, wm2_ref[...]) + bm2_ref[...], 0.0) for v in t]
        t = [jnp.maximum(dotb(v, wm3_ref[...]) + bm3_ref[...], 0.0) for v in t]
        seg = [jnp.dot(sc, v.astype(bf), preferred_element_type=f32)
               for v in t]
        m = [dotb(v, wm4_ref[...]) + bm4s_ref[...] for v in seg]

        gates = [jnp.dot(xb[p], wx_ref[...], preferred_element_type=f32)
                 + dotb(m[p], wm_ref[...]) for p in R]
        if step > 0:
            gates = [gates[p] + dotb(hs[p], wh_ref[...]) for p in R]
        i_ = [jax.nn.sigmoid(g[:, 0:H]) for g in gates]
        f_ = [jax.nn.sigmoid(g[:, 128:128 + H]) for g in gates]
        g_ = [jnp.tanh(g[:, 256:256 + H]) for g in gates]
        o_ = [jax.nn.sigmoid(g[:, 384:384 + H]) for g in gates]
        c = [f_[p] * c[p] + i_[p] * g_[p] for p in R]
        hs = [o_[p] * jnp.tanh(c[p]) for p in R]
        h = list(hs)

    mp = [jnp.max(v, axis=0, keepdims=True) for v in h]
    logit = [jnp.sum(h[p] * mp[p], axis=1, keepdims=True) * (1.0 / H)
             for p in R]
    logit = [v - jnp.max(v, axis=0, keepdims=True) for v in logit]
    ew = [jnp.exp(v) for v in logit]
    aw = [ew[p] / jnp.sum(ew[p], axis=0, keepdims=True) for p in R]
    pooled = [jnp.sum(h[p] * aw[p], axis=0, keepdims=True) for p in R]
    for p in R:
        out_ref[p] = dot(pooled[p], owt_ref[...]) + ob_ref[...]


def kernel(query, target, y_hat, digit_embed, row_embed, col_embed,
           in_layers, msg_layers, lstm_Wih, lstm_Whh, out_W, out_b,
           edge_index):
    f32 = jnp.float32
    prob = jnp.transpose(y_hat, (0, 2, 1))
    t_oh = jax.nn.one_hot(target, 10, dtype=f32)
    pin = jnp.concatenate([prob, t_oh], axis=-1)

    (w0, b0), (w1, b1), (w2, b2), (w3, b3) = in_layers
    m0 = jnp.concatenate([digit_embed @ w0[:, :EMB].T,
                          digit_embed @ w0[:, EMB:2 * EMB].T], axis=0)
    rows_e = jnp.asarray(_R_OH) @ row_embed
    cols_e = jnp.asarray(_C_OH) @ col_embed
    rc = (rows_e @ w0[:, 2 * EMB:3 * EMB].T
          + cols_e @ w0[:, 3 * EMB:].T + b0)

    bf = jnp.bfloat16
    (wm1, bm1), (wm2, bm2), (wm3, bm3), (wm4, bm4) = msg_layers
    wm1a = wm1[:, :H].T.astype(bf)
    wm1b = wm1[:, H:].T.astype(bf)
    bm4s = 20.0 * bm4

    wih_t = lstm_Wih.T
    wx = _pad_gates(wih_t[:H]).astype(bf)
    wm = _pad_gates(wih_t[H:]).astype(bf)
    wh = _pad_gates(lstm_Whh.T).astype(bf)

    row2 = lambda v: v[None, :]
    full = lambda arr: pl.BlockSpec(arr.shape, lambda b: (0,) * arr.ndim)

    ins = [pin, m0, rc,
           w1.T, row2(b1), w2.T, row2(b2), w3.T, row2(b3),
           jnp.asarray(_GS, dtype=bf), jnp.asarray(_SCT, dtype=bf),
           wm1a, wm1b, row2(bm1), wm2.T.astype(bf), row2(bm2),
           wm3.T.astype(bf), row2(bm3), wm4.T.astype(bf), row2(bm4s),
           wx, wm, wh, out_W.T, row2(out_b)]

    in_specs = [pl.BlockSpec((PP, N, 20), lambda b: (b, 0, 0))]
    in_specs += [full(a) for a in ins[1:]]

    out = pl.pallas_call(
        _fwd,
        grid=(B // PP,),
        in_specs=in_specs,
        out_specs=pl.BlockSpec((PP, 1, OUT), lambda b: (b, 0, 0)),
        out_shape=jax.ShapeDtypeStruct((B, 1, OUT), f32),
        compiler_params=pltpu.CompilerParams(
            dimension_semantics=("parallel",)),
    )(*ins)
    return out.reshape(B, OUT)


# speedup vs baseline: 2.8333x; 1.0942x over previous
import numpy as np
import jax
import jax.numpy as jnp
from jax.experimental import pallas as pl
from jax.experimental.pallas import tpu as pltpu

B = 128
EMB = 16
H = 96
STEPS = 4
OUT = 32
N = 81
E = 1620
EP = 1664


def _sudoku_edges():
    edges = set()
    for i in range(81):
        r, c = i // 9, i % 9
        for j in range(9):
            if j != c:
                edges.add((i, r * 9 + j))
            if j != r:
                edges.add((i, j * 9 + c))
        br, bc = 3 * (r // 3), 3 * (c // 3)
        for rr in range(br, br + 3):
            for cc in range(bc, bc + 3):
                j = rr * 9 + cc
                if j != i:
                    edges.add((i, j))
    e = np.array(sorted(edges), dtype=np.int32)
    return e[:, 0], e[:, 1]


_SRC, _DST = _sudoku_edges()
_PERM = np.lexsort((_SRC, _DST))
_SRC_D = _SRC[_PERM]

_GS = np.zeros((E, N), np.float32)
_GS[np.arange(E), _SRC_D] = 1.0
_SCT = np.zeros((N, E), np.float32)
_SCT[np.repeat(np.arange(N), 20), np.arange(E)] = 1.0

_CELL = np.arange(N)
_R_OH = np.zeros((N, 9), np.float32)
_R_OH[_CELL, _CELL // 9] = 1.0
_C_OH = np.zeros((N, 9), np.float32)
_C_OH[_CELL, _CELL % 9] = 1.0


def _pad_gates(w):
    parts = [jnp.pad(w[:, k * H:(k + 1) * H], ((0, 0), (0, 128 - H)))
             for k in range(4)]
    return jnp.concatenate(parts, axis=1)


PP = 8


def _fwd(pin_ref, m0_ref, rc_ref,
         w1_ref, b1_ref, w2_ref, b2_ref, w3_ref, b3_ref,
         gs_ref, sc_ref,
         wm1a_ref, wm1b_ref, bm1_ref, wm2_ref, bm2_ref, wm3_ref, bm3_ref,
         wm4_ref, bm4s_ref,
         wx_ref, wm_ref, wh_ref, owt_ref, ob_ref, out_ref):
    f32 = jnp.float32
    bf = jnp.bfloat16
    dot = lambda a, b: jnp.dot(a, b, preferred_element_type=f32)
    dotb = lambda a, b: jnp.dot(a.astype(bf), b, preferred_element_type=f32)
    R = range(PP)

    x = [pin_ref[p] for p in R]
    x = [jnp.maximum(dot(v, m0_ref[...]) + rc_ref[...], 0.0) for v in x]
    x = [jnp.maximum(dot(v, w1_ref[...]) + b1_ref[...], 0.0) for v in x]
    x = [jnp.maximum(dot(v, w2_ref[...]) + b2_ref[...], 0.0) for v in x]
    x = [dot(v, w3_ref[...]) + b3_ref[...] for v in x]

    gs = gs_ref[...]
    sc = sc_ref[...]

    h = list(x)
    xb = [v.astype(bf) for v in x]
    hs = [jnp.zeros((N, H), f32) for p in R]
    c = [jnp.zeros((N, H), f32) for p in R]
    for step in range(STEPS):
        hb = [v.astype(bf) for v in h]
        a = [dot(v, wm1a_ref[...]).astype(bf) for v in hb]
        bv = [dot(v, wm1b_ref[...]) + bm1_ref[...] for v in hb]
        t = [jnp.maximum(dot(gs, a[p]) + jnp.repeat(bv[p], 20, axis=0), 0.0)
             for p in R]
        t = [jnp.maximum(dotb(v, wm2_ref[...]) + bm2_ref[...], 0.0) for v in t]
        t = [jnp.maximum(dotb(v, wm3_ref[...]) + bm3_ref[...], 0.0) for v in t]
        seg = [jnp.dot(sc, v.astype(bf), preferred_element_type=f32)
               for v in t]
        m = [dotb(v, wm4_ref[...]) + bm4s_ref[...] for v in seg]

        gates = [jnp.dot(xb[p], wx_ref[...], preferred_element_type=f32)
                 + dotb(m[p], wm_ref[...]) for p in R]
        if step > 0:
            gates = [gates[p] + dotb(hs[p], wh_ref[...]) for p in R]
        i_ = [jax.nn.sigmoid(g[:, 0:H]) for g in gates]
        f_ = [jax.nn.sigmoid(g[:, 128:128 + H]) for g in gates]
        g_ = [jnp.tanh(g[:, 256:256 + H]) for g in gates]
        o_ = [jax.nn.sigmoid(g[:, 384:384 + H]) for g in gates]
        c = [f_[p] * c[p] + i_[p] * g_[p] for p in R]
        hs = [o_[p] * jnp.tanh(c[p]) for p in R]
        h = list(hs)

    mp = [jnp.max(v, axis=0, keepdims=True) for v in h]
    logit = [jnp.sum(h[p] * mp[p], axis=1, keepdims=True) * (1.0 / H)
             for p in R]
    logit = [v - jnp.max(v, axis=0, keepdims=True) for v in logit]
    ew = [jnp.exp(v) for v in logit]
    aw = [ew[p] / jnp.sum(ew[p], axis=0, keepdims=True) for p in R]
    pooled = [jnp.sum(h[p] * aw[p], axis=0, keepdims=True) for p in R]
    for p in R:
        out_ref[p] = dot(pooled[p], owt_ref[...]) + ob_ref[...]


def kernel(query, target, y_hat, digit_embed, row_embed, col_embed,
           in_layers, msg_layers, lstm_Wih, lstm_Whh, out_W, out_b,
           edge_index):
    f32 = jnp.float32
    prob = jnp.transpose(y_hat, (0, 2, 1))
    t_oh = jax.nn.one_hot(target, 10, dtype=f32)
    pin = jnp.concatenate([prob, t_oh], axis=-1)

    (w0, b0), (w1, b1), (w2, b2), (w3, b3) = in_layers
    m0 = jnp.concatenate([digit_embed @ w0[:, :EMB].T,
                          digit_embed @ w0[:, EMB:2 * EMB].T], axis=0)
    rows_e = jnp.asarray(_R_OH) @ row_embed
    cols_e = jnp.asarray(_C_OH) @ col_embed
    rc = (rows_e @ w0[:, 2 * EMB:3 * EMB].T
          + cols_e @ w0[:, 3 * EMB:].T + b0)

    bf = jnp.bfloat16
    (wm1, bm1), (wm2, bm2), (wm3, bm3), (wm4, bm4) = msg_layers
    wm1a = wm1[:, :H].T.astype(bf)
    wm1b = wm1[:, H:].T.astype(bf)
    bm4s = 20.0 * bm4

    wih_t = lstm_Wih.T
    wx = _pad_gates(wih_t[:H]).astype(bf)
    wm = _pad_gates(wih_t[H:]).astype(bf)
    wh = _pad_gates(lstm_Whh.T).astype(bf)

    row2 = lambda v: v[None, :]
    full = lambda arr: pl.BlockSpec(arr.shape, lambda b: (0,) * arr.ndim)

    ins = [pin, m0, rc,
           w1.T, row2(b1), w2.T, row2(b2), w3.T, row2(b3),
           jnp.asarray(_GS, dtype=bf), jnp.asarray(_SCT, dtype=bf),
           wm1a, wm1b, row2(bm1), wm2.T.astype(bf), row2(bm2),
           wm3.T.astype(bf), row2(bm3), wm4.T.astype(bf), row2(bm4s),
           wx, wm, wh, out_W.T, row2(out_b)]

    in_specs = [pl.BlockSpec((PP, N, 20), lambda b: (b, 0, 0))]
    in_specs += [full(a) for a in ins[1:]]

    out = pl.pallas_call(
        _fwd,
        grid=(B // PP,),
        in_specs=in_specs,
        out_specs=pl.BlockSpec((PP, 1, OUT), lambda b: (b, 0, 0)),
        out_shape=jax.ShapeDtypeStruct((B, 1, OUT), f32),
        compiler_params=pltpu.CompilerParams(
            dimension_semantics=("parallel",)),
    )(*ins)
    return out.reshape(B, OUT)
